# baseline (speedup 1.0000x reference)
import functools

import jax
import jax.numpy as jnp
from jax import lax
from jax.experimental import pallas as pl
from jax.experimental.pallas import tpu as pltpu
from jax.experimental.pallas import tpu_sc as plsc

_ROWS = 257
_D = 128

_mesh = plsc.ScalarSubcoreMesh(axis_name="c", num_cores=2)


@functools.partial(
    pl.kernel,
    mesh=_mesh,
    out_type=jax.ShapeDtypeStruct((_ROWS, _D), jnp.float32),
)
def _lookup(table_hbm, out_hbm):
    cid = lax.axis_index("c")

    @pl.when(cid == 0)
    def _lo():
        pltpu.sync_copy(table_hbm.at[pl.ds(0, 128)], out_hbm.at[pl.ds(0, 128)])

    @pl.when(cid == 1)
    def _hi():
        pltpu.sync_copy(table_hbm.at[pl.ds(128, 129)], out_hbm.at[pl.ds(128, 129)])


def kernel(seq_len, table):
    del seq_len
    return _lookup(table)
